# Initial kernel scaffold; baseline (speedup 1.0000x reference)
import jax
import jax.numpy as jnp
from jax.experimental import pallas as pl


def kernel(pod_id, pod_cpu, pod_mem, pod_location, pod_manifest, template_resource_id, template_cpu, template_mem, template_location, pod_table, template_table, pod_loc_table, template_loc_table, W_manifest, b_manifest, W_int, b_int, W1, b1, W2, b2, W3, b3):
    raise NotImplementedError("write your pallas kernel here")


# speedup vs baseline: 1.7731x; 1.7731x over previous
import functools

import jax
import jax.numpy as jnp
from jax import lax
from jax.experimental import pallas as pl
from jax.experimental.pallas import tpu as pltpu
from jax.experimental.pallas import tpu_sc as plsc

B = 16384
D = 32
BB = 2048


def _sc_gather4(tables, ids):
    info = plsc.get_sparse_core_info()
    nw = info.num_cores * info.num_subcores
    b_per_w = B // nw
    mesh = plsc.VectorSubcoreMesh(core_axis_name="c", subcore_axis_name="s")

    @functools.partial(
        pl.kernel,
        mesh=mesh,
        out_type=[jax.ShapeDtypeStruct((B, D), jnp.float32)] * 4,
        scratch_types=[
            pltpu.VMEM((b_per_w,), jnp.int32),
            pltpu.VMEM((b_per_w, D), jnp.float32),
            pltpu.SemaphoreType.DMA,
        ],
        compiler_params=pltpu.CompilerParams(use_tc_tiling_on_sc=False),
    )
    def gather_kernel(t0, t1, t2, t3, i0, i1, i2, i3,
                      o0, o1, o2, o3, idx_v, rows_v, sem):
        wid = lax.axis_index("s") * info.num_cores + lax.axis_index("c")
        base = wid * b_per_w
        for tab, idx, out in ((t0, i0, o0), (t1, i1, o1),
                              (t2, i2, o2), (t3, i3, o3)):
            pltpu.sync_copy(idx.at[pl.ds(base, b_per_w)], idx_v)
            pltpu.async_copy(tab.at[idx_v], rows_v, sem).wait()
            pltpu.sync_copy(rows_v, out.at[pl.ds(base, b_per_w)])

    return gather_kernel(*tables, *ids)


def _tc_body(cpu_f, mem_f, tcpu_f, tmem_f,
             cpu_c, mem_c, tcpu_c, tmem_c,
             manifest, e_pod, e_tmpl, e_ploc, e_tloc,
             W_manifest, b_manifest, W_int, b_int,
             W1, b1, W2, b2, W3, b3, out):
    eps = jnp.float32(1e-8)
    f32 = jnp.float32

    def norm(col, full):
        mn = jnp.min(full[...])
        mx = jnp.max(full[...])
        return (col[...] - mn) / (mx - mn + eps)

    w1 = W1[...]

    def slot(k):
        return w1[k * D:(k + 1) * D, :]

    def dot(a, b):
        return jax.lax.dot_general(a, b, (((1,), (0,)), ((), ())),
                                   preferred_element_type=f32)

    wi = W_int[...]
    bi = b_int[...]

    acc = dot(e_pod[...], slot(0))
    for k, (col, full) in zip((1, 2, 6, 7),
                              ((cpu_c, cpu_f), (mem_c, mem_f),
                               (tcpu_c, tcpu_f), (tmem_c, tmem_f))):
        s = slot(k)
        acc = acc + norm(col, full) * dot(wi, s) + dot(bi, s)
    acc = acc + dot(e_ploc[...], slot(3))
    m_emb = dot(manifest[...], W_manifest[...]) + b_manifest[...]
    acc = acc + dot(m_emb, slot(4))
    acc = acc + dot(e_tmpl[...], slot(5))
    acc = acc + dot(e_tloc[...], slot(8))
    acc = acc + b1[...]

    h1 = jnp.maximum(acc, 0.0)
    h2 = jnp.maximum(dot(h1, W2[...]) + b2[...], 0.0)
    out[...] = dot(h2, W3[...]) + b3[...]


def _tc_forward(cpu, mem, tcpu, tmem, manifest,
                e_pod, e_tmpl, e_ploc, e_tloc,
                W_manifest, b_manifest, W_int, b_int,
                W1, b1, W2, b2, W3, b3, interpret=False):
    grid = (B // BB,)
    full2 = lambda shape: pl.BlockSpec(shape, lambda i: (0, 0))
    blk = lambda shape: pl.BlockSpec(shape, lambda i: (i, 0))
    in_specs = [
        full2((1, B)), full2((1, B)), full2((1, B)), full2((1, B)),
        blk((BB, 1)), blk((BB, 1)), blk((BB, 1)), blk((BB, 1)),
        blk((BB, 512)),
        blk((BB, D)), blk((BB, D)), blk((BB, D)), blk((BB, D)),
        full2((512, D)), full2((1, D)), full2((1, D)), full2((1, D)),
        full2((9 * D, 256)), full2((1, 256)),
        full2((256, 64)), full2((1, 64)),
        full2((64, 1)), full2((1, 1)),
    ]
    return pl.pallas_call(
        _tc_body,
        grid=grid,
        in_specs=in_specs,
        out_specs=blk((BB, 1)),
        out_shape=jax.ShapeDtypeStruct((B, 1), jnp.float32),
        compiler_params=pltpu.CompilerParams(
            dimension_semantics=("arbitrary",)),
        interpret=interpret,
    )(cpu.reshape(1, B), mem.reshape(1, B), tcpu.reshape(1, B),
      tmem.reshape(1, B),
      cpu.reshape(B, 1), mem.reshape(B, 1), tcpu.reshape(B, 1),
      tmem.reshape(B, 1),
      manifest, e_pod, e_tmpl, e_ploc, e_tloc,
      W_manifest, b_manifest.reshape(1, D), W_int, b_int.reshape(1, D),
      W1, b1.reshape(1, 256), W2, b2.reshape(1, 64),
      W3, b3.reshape(1, 1))


def kernel(pod_id, pod_cpu, pod_mem, pod_location, pod_manifest,
           template_resource_id, template_cpu, template_mem,
           template_location, pod_table, template_table, pod_loc_table,
           template_loc_table, W_manifest, b_manifest, W_int, b_int,
           W1, b1, W2, b2, W3, b3):
    i32 = jnp.int32
    e_pod, e_tmpl, e_ploc, e_tloc = _sc_gather4(
        (pod_table, template_table, pod_loc_table, template_loc_table),
        (pod_id.astype(i32), template_resource_id.astype(i32),
         pod_location.astype(i32), template_location.astype(i32)))
    return _tc_forward(pod_cpu, pod_mem, template_cpu, template_mem,
                       pod_manifest, e_pod, e_tmpl, e_ploc, e_tloc,
                       W_manifest, b_manifest, W_int, b_int,
                       W1, b1, W2, b2, W3, b3)


# speedup vs baseline: 2.3698x; 1.3365x over previous
import functools

import jax
import jax.numpy as jnp
from jax import lax
from jax.experimental import pallas as pl
from jax.experimental.pallas import tpu as pltpu
from jax.experimental.pallas import tpu_sc as plsc

B = 16384
D = 32
BB = 2048


def _sc_gather4(tables, ids):
    info = plsc.get_sparse_core_info()
    nw = info.num_cores * info.num_subcores
    b_per_w = B // nw
    mesh = plsc.VectorSubcoreMesh(core_axis_name="c", subcore_axis_name="s")

    @functools.partial(
        pl.kernel,
        mesh=mesh,
        out_type=jax.ShapeDtypeStruct((B, 4 * D), jnp.float32),
        scratch_types=[
            pltpu.VMEM((4, b_per_w), jnp.int32),
            pltpu.VMEM((4, b_per_w, D), jnp.float32),
            pltpu.SemaphoreType.DMA,
        ],
        compiler_params=pltpu.CompilerParams(use_tc_tiling_on_sc=False),
    )
    def gather_kernel(t0, t1, t2, t3, i0, i1, i2, i3,
                      e_out, idx_v, rows_v, sem):
        wid = lax.axis_index("s") * info.num_cores + lax.axis_index("c")
        base = wid * b_per_w
        tabs = (t0, t1, t2, t3)
        for t, idx in enumerate((i0, i1, i2, i3)):
            pltpu.sync_copy(idx.at[pl.ds(base, b_per_w)], idx_v.at[t])
        copies = [
            pltpu.async_copy(tabs[t].at[idx_v.at[t]], rows_v.at[t], sem)
            for t in range(4)
        ]
        for t in range(4):
            copies[t].wait()
            pltpu.sync_copy(
                rows_v.at[t],
                e_out.at[pl.ds(base, b_per_w), pl.ds(t * D, D)])

    return gather_kernel(*tables, *ids)


def _tc_body(cpu_f, mem_f, tcpu_f, tmem_f,
             cpu_c, mem_c, tcpu_c, tmem_c,
             manifest, emb,
             W_manifest, b_manifest, W_int, b_int,
             W1, W1sel, b1, W2, b2, W3, b3, out):
    eps = jnp.float32(1e-8)
    f32 = jnp.float32
    bf16 = jnp.bfloat16

    def norm(col, full):
        mn = jnp.min(full[...])
        mx = jnp.max(full[...])
        return (col[...] - mn) / (mx - mn + eps)

    w1 = W1[...]

    def slot(k):
        return w1[k * D:(k + 1) * D, :]

    def dot(a, b):
        return jax.lax.dot_general(a, b, (((1,), (0,)), ((), ())),
                                   preferred_element_type=f32)

    def bdot(a, b):
        return dot(a.astype(bf16), b.astype(bf16))

    wi = W_int[...]
    bi = b_int[...]

    acc = bdot(emb[...], W1sel[...])
    for k, (col, full) in zip((1, 2, 6, 7),
                              ((cpu_c, cpu_f), (mem_c, mem_f),
                               (tcpu_c, tcpu_f), (tmem_c, tmem_f))):
        s = slot(k)
        acc = acc + norm(col, full) * dot(wi, s) + dot(bi, s)
    s4 = slot(4)
    m_fold = bdot(W_manifest[...], s4)
    acc = acc + bdot(manifest[...], m_fold)
    acc = acc + dot(b_manifest[...], s4)
    acc = acc + b1[...]

    h1 = jnp.maximum(acc, 0.0)
    h2 = jnp.maximum(bdot(h1, W2[...]) + b2[...], 0.0)
    out[...] = bdot(h2, W3[...]) + b3[...]


def _tc_forward(cpu, mem, tcpu, tmem, manifest, emb,
                W_manifest, b_manifest, W_int, b_int,
                W1, b1, W2, b2, W3, b3, interpret=False):
    grid = (B // BB,)
    full2 = lambda shape: pl.BlockSpec(shape, lambda i: (0, 0))
    blk = lambda shape: pl.BlockSpec(shape, lambda i: (i, 0))
    W1sel = jnp.concatenate(
        [W1[0 * D:1 * D], W1[3 * D:4 * D], W1[5 * D:6 * D], W1[8 * D:9 * D]],
        axis=0)
    in_specs = [
        full2((1, B)), full2((1, B)), full2((1, B)), full2((1, B)),
        blk((BB, 1)), blk((BB, 1)), blk((BB, 1)), blk((BB, 1)),
        blk((BB, 512)),
        blk((BB, 4 * D)),
        full2((512, D)), full2((1, D)), full2((1, D)), full2((1, D)),
        full2((9 * D, 256)), full2((4 * D, 256)), full2((1, 256)),
        full2((256, 64)), full2((1, 64)),
        full2((64, 1)), full2((1, 1)),
    ]
    return pl.pallas_call(
        _tc_body,
        grid=grid,
        in_specs=in_specs,
        out_specs=blk((BB, 1)),
        out_shape=jax.ShapeDtypeStruct((B, 1), jnp.float32),
        compiler_params=pltpu.CompilerParams(
            dimension_semantics=("arbitrary",)),
        interpret=interpret,
    )(cpu.reshape(1, B), mem.reshape(1, B), tcpu.reshape(1, B),
      tmem.reshape(1, B),
      cpu.reshape(B, 1), mem.reshape(B, 1), tcpu.reshape(B, 1),
      tmem.reshape(B, 1),
      manifest, emb,
      W_manifest, b_manifest.reshape(1, D), W_int, b_int.reshape(1, D),
      W1, W1sel, b1.reshape(1, 256), W2, b2.reshape(1, 64),
      W3, b3.reshape(1, 1))


def kernel(pod_id, pod_cpu, pod_mem, pod_location, pod_manifest,
           template_resource_id, template_cpu, template_mem,
           template_location, pod_table, template_table, pod_loc_table,
           template_loc_table, W_manifest, b_manifest, W_int, b_int,
           W1, b1, W2, b2, W3, b3):
    i32 = jnp.int32
    emb = _sc_gather4(
        (pod_table, pod_loc_table, template_table, template_loc_table),
        (pod_id.astype(i32), pod_location.astype(i32),
         template_resource_id.astype(i32), template_location.astype(i32)))
    return _tc_forward(pod_cpu, pod_mem, template_cpu, template_mem,
                       pod_manifest, emb,
                       W_manifest, b_manifest, W_int, b_int,
                       W1, b1, W2, b2, W3, b3)
